# Initial kernel scaffold; baseline (speedup 1.0000x reference)
import jax
import jax.numpy as jnp
from jax.experimental import pallas as pl


def kernel(codes, codebooks, W, b):
    raise NotImplementedError("write your pallas kernel here")


# speedup vs baseline: 6.4658x; 6.4658x over previous
import functools

import jax
import jax.numpy as jnp
from jax import lax
from jax.experimental import pallas as pl
from jax.experimental.pallas import tpu as pltpu
from jax.experimental.pallas import tpu_sc as plsc

NUM_Q = 8
CODEBOOK_SIZE = 1024
DIM = 64
HIDDEN = 1024
BATCH = 16
TLEN = 2048
NTOK = BATCH * TLEN
KDIM = NUM_Q * DIM

NC = 2
NS = 16
NW = NC * NS
TOK_PER_W = NTOK // NW
CHUNK = 64
NCHUNK = TOK_PER_W // CHUNK


def _sc_gather(codes_flat, cb_flat):
    mesh = plsc.VectorSubcoreMesh(
        core_axis_name="c", subcore_axis_name="s", num_cores=NC, num_subcores=NS
    )

    @functools.partial(
        pl.kernel,
        mesh=mesh,
        out_type=jax.ShapeDtypeStruct((NTOK, KDIM), jnp.float32),
        scratch_types=[
            pltpu.VMEM((NUM_Q, CHUNK), jnp.int32),
            pltpu.VMEM((NUM_Q, CHUNK, DIM), jnp.float32),
            pltpu.SemaphoreType.DMA,
        ],
        compiler_params=pltpu.CompilerParams(use_tc_tiling_on_sc=False),
    )
    def k(codes_hbm, cb_hbm, q_hbm, idx_v, dst_v, sem):
        wid = lax.axis_index("s") * NC + lax.axis_index("c")
        wbase = wid * TOK_PER_W

        def chunk_body(ci, carry):
            base = wbase + ci * CHUNK
            pltpu.sync_copy(codes_hbm.at[:, pl.ds(base, CHUNK)], idx_v)
            for i in range(1, NUM_Q):
                for j in range(CHUNK // 16):
                    sl = pl.ds(j * 16, 16)
                    idx_v[i, sl] = idx_v[i, sl] + (i * CODEBOOK_SIZE)
            copies = [
                pltpu.async_copy(
                    cb_hbm.at[idx_v.at[i]],
                    dst_v.at[i],
                    sem,
                )
                for i in range(NUM_Q)
            ]
            for cp in copies:
                cp.wait()
            for i in range(NUM_Q):
                pltpu.sync_copy(
                    dst_v.at[i],
                    q_hbm.at[pl.ds(base, CHUNK), pl.ds(i * DIM, DIM)],
                )
            return carry

        lax.fori_loop(0, NCHUNK, chunk_body, 0)

    return k(codes_flat, cb_flat)


TB = 512


def _tc_matmul_body(q_ref, wt_ref, bt_ref, out_ref):
    qb = q_ref[...].astype(jnp.bfloat16)
    acc = lax.dot_general(
        wt_ref[...], qb,
        dimension_numbers=(((1,), (1,)), ((), ())),
        preferred_element_type=jnp.float32,
    )
    bsum = jnp.sum(bt_ref[...], axis=1, keepdims=True)
    out_ref[0, :, :] = acc + bsum


def _tc_matmul(q, w_t, b_t):
    grid = (BATCH, TLEN // TB)
    return pl.pallas_call(
        _tc_matmul_body,
        grid=grid,
        in_specs=[
            pl.BlockSpec((TB, KDIM), lambda bi, ti: (bi * (TLEN // TB) + ti, 0)),
            pl.BlockSpec((HIDDEN, KDIM), lambda bi, ti: (0, 0)),
            pl.BlockSpec((HIDDEN, NUM_Q), lambda bi, ti: (0, 0)),
        ],
        out_specs=pl.BlockSpec((1, HIDDEN, TB), lambda bi, ti: (bi, 0, ti)),
        out_shape=jax.ShapeDtypeStruct((BATCH, HIDDEN, TLEN), jnp.float32),
        compiler_params=pltpu.CompilerParams(
            dimension_semantics=("parallel", "parallel"),
        ),
    )(q, w_t, b_t)


def kernel(codes, codebooks, W, b):
    codes_flat = codes.astype(jnp.int32).reshape(NUM_Q, NTOK)
    cb_flat = codebooks.reshape(NUM_Q * CODEBOOK_SIZE, DIM)
    w_t = jnp.transpose(W.reshape(KDIM, HIDDEN)).astype(jnp.bfloat16)
    b_t = jnp.transpose(b)
    q = _sc_gather(codes_flat, cb_flat)
    return _tc_matmul(q, w_t, b_t)
